# baseline (speedup 1.0000x reference)
import functools

import jax
import jax.numpy as jnp
import numpy as np
from jax import lax
from jax.experimental import pallas as pl
from jax.experimental.pallas import tpu as pltpu
from jax.experimental.pallas import tpu_sc as plsc

N = 10000
D = 128
NC, NS = 2, 16
NW = NC * NS
N_ACC = 10016
ROWS_Z = N_ACC // NS
ROWS_O = 624
DUMMY = N
NBUF = 4
GOFF = 2
SOFF = 3

CH = 88
P1 = (152, 76)
P2 = (56, 16)


def _make_seg(npt0, npt1, ch):
    mesh = plsc.VectorSubcoreMesh(
        core_axis_name="c", subcore_axis_name="s", num_cores=NC, num_subcores=NS
    )

    @functools.partial(
        pl.kernel,
        out_type=jax.ShapeDtypeStruct((NC, N, D), jnp.float32),
        mesh=mesh,
        scratch_types=[
            pltpu.VMEM_SHARED((N_ACC, D), jnp.float32)
        ]
        + [pltpu.VMEM((ch, D), jnp.float32)] * NBUF
        + [pltpu.VMEM((ch,), jnp.int32)] * NBUF
        + [pltpu.VMEM((ch,), jnp.int32)] * NBUF
        + [pltpu.SemaphoreType.DMA] * (3 * NBUF),
    )
    def seg(table, gidx, sidx, out, acc, *bufs):
        rb = bufs[0 * NBUF:1 * NBUF]
        gib = bufs[1 * NBUF:2 * NBUF]
        sib = bufs[2 * NBUF:3 * NBUF]
        isem = bufs[3 * NBUF:4 * NBUF]
        gsem = bufs[4 * NBUF:5 * NBUF]
        ssem = bufs[5 * NBUF:6 * NBUF]
        c = lax.axis_index("c")
        s = lax.axis_index("s")

        def scatter_issue(b):
            pltpu.make_async_copy(table.at[gib[b]], rb[b], gsem[b]).wait()
            pltpu.async_copy(rb[b], acc.at[sib[b]], ssem[b], add=True)

        def scatter_drain(b):
            pltpu.make_async_copy(rb[b], acc.at[sib[b]], ssem[b]).wait()

        z = jnp.zeros((16,), jnp.float32)

        def zb(i, carry):
            for j in range(D // 16):
                rb[0][i, pl.ds(j * 16, 16)] = z
            return carry

        lax.fori_loop(0, ch, zb, 0)
        zbase = s * ROWS_Z
        rem = ROWS_Z % ch

        def zero_descs():
            for k in range(ROWS_Z // ch):
                yield rb[0], acc.at[pl.ds(zbase + k * ch, ch)]
            if rem:
                yield rb[0].at[pl.ds(0, rem)], acc.at[
                    pl.ds(zbase + (ROWS_Z // ch) * ch, rem)
                ]

        for src, dst in zero_descs():
            pltpu.async_copy(src, dst, isem[0])
        for src, dst in zero_descs():
            pltpu.make_async_copy(src, dst, isem[0]).wait()
        plsc.subcore_barrier()

        def run(npt, base):
            def idx_issue(t, b):
                off = (base + t) * ch
                pltpu.async_copy(gidx.at[pl.ds(off, ch)], gib[b], isem[b])
                pltpu.async_copy(sidx.at[pl.ds(off, ch)], sib[b], isem[b])

            def gather_issue(t, b):
                off = (base + t) * ch
                pltpu.make_async_copy(gidx.at[pl.ds(off, ch)], gib[b], isem[b]).wait()
                pltpu.make_async_copy(sidx.at[pl.ds(off, ch)], sib[b], isem[b]).wait()
                pltpu.async_copy(table.at[gib[b]], rb[b], gsem[b])

            def body(j, carry):
                for b in range(NBUF):
                    i = j * NBUF + b

                    @pl.when(j > 0)
                    def _free():
                        scatter_drain(b)

                    idx_issue(i, b)

                    @pl.when(i >= GOFF)
                    def _g():
                        gather_issue(i - GOFF, (b - GOFF) % NBUF)

                    @pl.when(i >= SOFF)
                    def _s():
                        scatter_issue((b - SOFF) % NBUF)

                return carry

            lax.fori_loop(0, npt // NBUF, body, 0)
            for t in range(npt - GOFF, npt):
                gather_issue(t, t % NBUF)
            for t in range(npt - SOFF, npt):
                scatter_drain((t - 1) % NBUF)
                scatter_issue(t % NBUF)
            scatter_drain((npt - 1) % NBUF)

        @pl.when(c == 0)
        def _c0():
            run(npt0, s * npt0)

        @pl.when(c == 1)
        def _c1():
            run(npt1, NS * npt0 + s * npt1)

        plsc.subcore_barrier()

        obase = s * ROWS_O
        pltpu.sync_copy(acc.at[pl.ds(obase, ROWS_O)], out.at[c, pl.ds(obase, ROWS_O)])

        @pl.when(s == NS - 1)
        def _tail():
            tb = NS * ROWS_O
            pltpu.sync_copy(acc.at[pl.ds(tb, N - tb)], out.at[c, pl.ds(tb, N - tb)])

    return seg


_seg_neighbor = _make_seg(P1[0], P1[1], CH)
_seg_sub = _make_seg(P2[0], P2[1], CH)


BM = 2000


def _mm_a_body(x_ref, p_ref, wr_ref, wn_ref, b_ref, o_ref):
    agg = p_ref[0] + p_ref[1]
    o_ref[...] = (
        jnp.dot(x_ref[...], wr_ref[...], preferred_element_type=jnp.float32)
        + jnp.dot(agg, wn_ref[...], preferred_element_type=jnp.float32)
        + b_ref[...]
    )


def _mm_b_body(p_ref, w1_ref, b1_ref, w2_ref, o_ref):
    t = (
        jnp.dot(p_ref[0] + p_ref[1], w1_ref[...], preferred_element_type=jnp.float32)
        + b1_ref[...]
    )
    o_ref[...] = jnp.dot(t, w2_ref[...], preferred_element_type=jnp.float32)


def _mm_c_body(h_ref, q_ref, b2_ref, o_ref):
    o_ref[...] = h_ref[...] + q_ref[0] + q_ref[1] + b2_ref[...]


_ROW = pl.BlockSpec((BM, D), lambda i: (i, 0))
_PART = pl.BlockSpec((NC, BM, D), lambda i: (0, i, 0))
_WMAT = pl.BlockSpec((D, D), lambda i: (0, 0))
_BVEC = pl.BlockSpec((1, D), lambda i: (0, 0))
_OUTF = jax.ShapeDtypeStruct((N, D), jnp.float32)


def _mm_a(x, p, wr, wn, b):
    return pl.pallas_call(
        _mm_a_body,
        grid=(N // BM,),
        in_specs=[_ROW, _PART, _WMAT, _WMAT, _BVEC],
        out_specs=_ROW,
        out_shape=_OUTF,
    )(x, p, wr, wn, b)


def _mm_b(p, w1, b1, w2):
    return pl.pallas_call(
        _mm_b_body,
        grid=(N // BM,),
        in_specs=[_PART, _WMAT, _BVEC, _WMAT],
        out_specs=_ROW,
        out_shape=_OUTF,
    )(p, w1, b1, w2)


def _mm_c(h, q, b2):
    return pl.pallas_call(
        _mm_c_body,
        grid=(N // BM,),
        in_specs=[_ROW, _PART, _BVEC],
        out_specs=_ROW,
        out_shape=_OUTF,
    )(h, q, b2)


_PAD_G = {}
_PAD_S = {}
for _npts, _e in ((P1, N_EDGES_1 := 320000), (P2, N_EDGES_2 := 100000)):
    _pad = NS * (_npts[0] + _npts[1]) * CH - _e
    _PAD_G[_npts] = np.zeros((_pad,), np.int32)
    _PAD_S[_npts] = (DUMMY + np.arange(_pad, dtype=np.int32) % NS).astype(np.int32)


def _pad_edges(g, sc, npts):
    g2 = jnp.concatenate([g, _PAD_G[npts]])
    s2 = jnp.concatenate([sc, _PAD_S[npts]])
    return g2, s2


def kernel(x, neighbor_edge_index, substructures_edge_index, W_root, W_nb, b_mn, W_n2s, b_n2s, W_s2n, b_s2n):
    src = neighbor_edge_index[0]
    dst = neighbor_edge_index[1]
    sei = substructures_edge_index[0]
    row = sei[0]
    col = sei[1]

    g1, s1 = _pad_edges(src, dst, P1)
    g2, s2 = _pad_edges(row, col, P2)
    g3, s3 = _pad_edges(col, row, P2)

    b_mn2 = b_mn.reshape(1, D)
    b_n2s2 = b_n2s.reshape(1, D)
    b_s2n2 = b_s2n.reshape(1, D)

    agg = _seg_neighbor(x, g1, s1)
    h = _mm_a(x, agg, W_root, W_nb, b_mn2)
    sub = _seg_sub(h, g2, s2)
    t2 = _mm_b(sub, W_n2s, b_n2s2, W_s2n)
    q = _seg_sub(t2, g3, s3)
    return _mm_c(h, q, b_s2n2)


# speedup vs baseline: 1.0673x; 1.0673x over previous
import functools

import jax
import jax.numpy as jnp
import numpy as np
from jax import lax
from jax.experimental import pallas as pl
from jax.experimental.pallas import tpu as pltpu
from jax.experimental.pallas import tpu_sc as plsc

N = 10000
D = 128
NC, NS = 2, 16
NW = NC * NS
N_ACC = 10016
ROWS_Z = N_ACC // NS
ROWS_O = 624
DUMMY = N
NBUF = 4
GOFF = 2
SOFF = 3

CH = 88
P1 = (132, 96)
P2 = (56, 16)


def _make_seg(npt0, npt1, ch):
    mesh = plsc.VectorSubcoreMesh(
        core_axis_name="c", subcore_axis_name="s", num_cores=NC, num_subcores=NS
    )

    @functools.partial(
        pl.kernel,
        out_type=jax.ShapeDtypeStruct((NC, N, D), jnp.float32),
        mesh=mesh,
        scratch_types=[
            pltpu.VMEM_SHARED((N_ACC, D), jnp.float32)
        ]
        + [pltpu.VMEM((ch, D), jnp.float32)] * NBUF
        + [pltpu.VMEM((ch,), jnp.int32)] * NBUF
        + [pltpu.VMEM((ch,), jnp.int32)] * NBUF
        + [pltpu.SemaphoreType.DMA] * (3 * NBUF),
    )
    def seg(table, edges, out, acc, *bufs):
        rb = bufs[0 * NBUF:1 * NBUF]
        gib = bufs[1 * NBUF:2 * NBUF]
        sib = bufs[2 * NBUF:3 * NBUF]
        isem = bufs[3 * NBUF:4 * NBUF]
        gsem = bufs[4 * NBUF:5 * NBUF]
        ssem = bufs[5 * NBUF:6 * NBUF]
        c = lax.axis_index("c")
        s = lax.axis_index("s")

        def scatter_issue(b):
            pltpu.make_async_copy(table.at[gib[b]], rb[b], gsem[b]).wait()
            pltpu.async_copy(rb[b], acc.at[sib[b]], ssem[b], add=True)

        def scatter_drain(b):
            pltpu.make_async_copy(rb[b], acc.at[sib[b]], ssem[b]).wait()

        z = jnp.zeros((16,), jnp.float32)

        def zb(i, carry):
            for j in range(D // 16):
                rb[0][i, pl.ds(j * 16, 16)] = z
            return carry

        lax.fori_loop(0, ch, zb, 0)
        zbase = s * ROWS_Z
        rem = ROWS_Z % ch

        def zero_descs():
            for k in range(ROWS_Z // ch):
                yield rb[0], acc.at[pl.ds(zbase + k * ch, ch)]
            if rem:
                yield rb[0].at[pl.ds(0, rem)], acc.at[
                    pl.ds(zbase + (ROWS_Z // ch) * ch, rem)
                ]

        for src, dst in zero_descs():
            pltpu.async_copy(src, dst, isem[0])
        for src, dst in zero_descs():
            pltpu.make_async_copy(src, dst, isem[0]).wait()
        plsc.subcore_barrier()

        cap = NS * (npt0 + npt1) * ch
        npt = jnp.where(c == 0, npt0, npt1)
        base = jnp.where(c == 0, s * npt0, NS * npt0 + s * npt1)

        def idx_issue(t, b):
            off = (base + t) * ch
            pltpu.async_copy(edges.at[pl.ds(off, ch)], gib[b], isem[b])
            pltpu.async_copy(edges.at[pl.ds(cap + off, ch)], sib[b], isem[b])

        def gather_issue(t, b):
            off = (base + t) * ch
            pltpu.make_async_copy(edges.at[pl.ds(off, ch)], gib[b], isem[b]).wait()
            pltpu.make_async_copy(edges.at[pl.ds(cap + off, ch)], sib[b], isem[b]).wait()
            pltpu.async_copy(table.at[gib[b]], rb[b], gsem[b])

        def body(j, carry):
            for b in range(NBUF):
                i = j * NBUF + b

                @pl.when(j > 0)
                def _free():
                    scatter_drain(b)

                idx_issue(i, b)

                @pl.when(i >= GOFF)
                def _g():
                    gather_issue(i - GOFF, (b - GOFF) % NBUF)

                @pl.when(i >= SOFF)
                def _s():
                    scatter_issue((b - SOFF) % NBUF)

            return carry

        lax.fori_loop(0, npt // NBUF, body, 0)
        for k in range(GOFF):
            gather_issue(npt - GOFF + k, (NBUF - GOFF + k) % NBUF)
        for k in range(SOFF):
            scatter_drain((NBUF - SOFF + k - 1) % NBUF)
            scatter_issue((NBUF - SOFF + k) % NBUF)
        scatter_drain(NBUF - 1)

        plsc.subcore_barrier()

        obase = s * ROWS_O
        pltpu.sync_copy(acc.at[pl.ds(obase, ROWS_O)], out.at[c, pl.ds(obase, ROWS_O)])

        @pl.when(s == NS - 1)
        def _tail():
            tb = NS * ROWS_O
            pltpu.sync_copy(acc.at[pl.ds(tb, N - tb)], out.at[c, pl.ds(tb, N - tb)])

    return seg


_seg_neighbor = _make_seg(P1[0], P1[1], CH)
_seg_sub = _make_seg(P2[0], P2[1], CH)


BM = 2000


def _mm_a_body(x_ref, p_ref, wr_ref, wn_ref, b_ref, o_ref):
    agg = p_ref[0] + p_ref[1]
    o_ref[...] = (
        jnp.dot(x_ref[...], wr_ref[...], preferred_element_type=jnp.float32)
        + jnp.dot(agg, wn_ref[...], preferred_element_type=jnp.float32)
        + b_ref[...]
    )


def _mm_b_body(p_ref, w1_ref, b1_ref, w2_ref, o_ref):
    t = (
        jnp.dot(p_ref[0] + p_ref[1], w1_ref[...], preferred_element_type=jnp.float32)
        + b1_ref[...]
    )
    o_ref[...] = jnp.dot(t, w2_ref[...], preferred_element_type=jnp.float32)


def _mm_c_body(h_ref, q_ref, b2_ref, o_ref):
    o_ref[...] = h_ref[...] + q_ref[0] + q_ref[1] + b2_ref[...]


_ROW = pl.BlockSpec((BM, D), lambda i: (i, 0))
_PART = pl.BlockSpec((NC, BM, D), lambda i: (0, i, 0))
_WMAT = pl.BlockSpec((D, D), lambda i: (0, 0))
_BVEC = pl.BlockSpec((1, D), lambda i: (0, 0))
_OUTF = jax.ShapeDtypeStruct((N, D), jnp.float32)


def _mm_a(x, p, wr, wn, b):
    return pl.pallas_call(
        _mm_a_body,
        grid=(N // BM,),
        in_specs=[_ROW, _PART, _WMAT, _WMAT, _BVEC],
        out_specs=_ROW,
        out_shape=_OUTF,
    )(x, p, wr, wn, b)


def _mm_b(p, w1, b1, w2):
    return pl.pallas_call(
        _mm_b_body,
        grid=(N // BM,),
        in_specs=[_PART, _WMAT, _BVEC, _WMAT],
        out_specs=_ROW,
        out_shape=_OUTF,
    )(p, w1, b1, w2)


def _mm_c(h, q, b2):
    return pl.pallas_call(
        _mm_c_body,
        grid=(N // BM,),
        in_specs=[_ROW, _PART, _BVEC],
        out_specs=_ROW,
        out_shape=_OUTF,
    )(h, q, b2)


def _pad_const(npts, n_edges):
    pad = NS * (npts[0] + npts[1]) * CH - n_edges
    return np.stack(
        [np.zeros((pad,), np.int32),
         (DUMMY + np.arange(pad, dtype=np.int32) % NS).astype(np.int32)]
    )


_PADN = _pad_const(P1, 320000)
_PADS = _pad_const(P2, 100000)


def kernel(x, neighbor_edge_index, substructures_edge_index, W_root, W_nb, b_mn, W_n2s, b_n2s, W_s2n, b_s2n):
    sei = substructures_edge_index[0]
    e1 = jnp.concatenate([neighbor_edge_index, _PADN], axis=1).reshape(-1)
    e2 = jnp.concatenate([sei, _PADS], axis=1).reshape(-1)
    e3 = jnp.concatenate([sei[::-1], _PADS], axis=1).reshape(-1)

    b_mn2 = b_mn.reshape(1, D)
    b_n2s2 = b_n2s.reshape(1, D)
    b_s2n2 = b_s2n.reshape(1, D)

    agg = _seg_neighbor(x, e1)
    h = _mm_a(x, agg, W_root, W_nb, b_mn2)
    sub = _seg_sub(h, e2)
    t2 = _mm_b(sub, W_n2s, b_n2s2, W_s2n)
    q = _seg_sub(t2, e3)
    return _mm_c(h, q, b_s2n2)


# speedup vs baseline: 1.2763x; 1.1958x over previous
import functools

import jax
import jax.numpy as jnp
import numpy as np
from jax import lax
from jax.experimental import pallas as pl
from jax.experimental.pallas import tpu as pltpu
from jax.experimental.pallas import tpu_sc as plsc

N = 10000
D = 128
NC, NS = 2, 16
NW = NC * NS
N_ACC = 10016
ROWS_Z = N_ACC // NS
ROWS_O = 624
DUMMY = N
NBUF = 4
GOFF = 2
SOFF = 3

CH = 88
P1 = (132, 96)
P2 = (56, 16)


def _make_seg(npt0, npt1, ch, nt, swap=False):
    mesh = plsc.VectorSubcoreMesh(
        core_axis_name="c", subcore_axis_name="s", num_cores=NC, num_subcores=NS
    )

    @functools.partial(
        pl.kernel,
        out_type=jax.ShapeDtypeStruct((NC, N, D), jnp.float32),
        mesh=mesh,
        scratch_types=[
            pltpu.VMEM_SHARED((N_ACC, D), jnp.float32)
        ]
        + [pltpu.VMEM((ch, D), jnp.float32)] * NBUF
        + [pltpu.VMEM((ch,), jnp.int32)] * NBUF
        + [pltpu.VMEM((ch,), jnp.int32)] * NBUF
        + [pltpu.SemaphoreType.DMA] * (3 * NBUF),
    )
    def seg(table, edges, out, acc, *bufs):
        rb = bufs[0 * NBUF:1 * NBUF]
        gib = bufs[1 * NBUF:2 * NBUF]
        sib = bufs[2 * NBUF:3 * NBUF]
        isem = bufs[3 * NBUF:4 * NBUF]
        gsem = bufs[4 * NBUF:5 * NBUF]
        ssem = bufs[5 * NBUF:6 * NBUF]
        c = lax.axis_index("c")
        s = lax.axis_index("s")

        def scatter_issue(b):
            pltpu.make_async_copy(table.at[gib[b]], rb[b], gsem[b]).wait()
            pltpu.async_copy(rb[b], acc.at[sib[b]], ssem[b], add=True)

        def scatter_drain(b):
            pltpu.make_async_copy(rb[b], acc.at[sib[b]], ssem[b]).wait()

        z = jnp.zeros((16,), jnp.float32)

        def zb(i, carry):
            for j in range(D // 16):
                rb[0][i, pl.ds(j * 16, 16)] = z
            return carry

        lax.fori_loop(0, ch, zb, 0)
        zbase = s * ROWS_Z
        rem = ROWS_Z % ch

        def zero_descs():
            for k in range(ROWS_Z // ch):
                yield rb[0], acc.at[pl.ds(zbase + k * ch, ch)]
            if rem:
                yield rb[0].at[pl.ds(0, rem)], acc.at[
                    pl.ds(zbase + (ROWS_Z // ch) * ch, rem)
                ]

        for src, dst in zero_descs():
            pltpu.async_copy(src, dst, isem[0])
        for src, dst in zero_descs():
            pltpu.make_async_copy(src, dst, isem[0]).wait()
        plsc.subcore_barrier()

        cap = NS * (npt0 + npt1) * ch
        gbase, sbase = (cap, 0) if swap else (0, cap)
        npt = jnp.where(c == 0, npt0, npt1)
        base = jnp.where(c == 0, s * npt0, NS * npt0 + s * npt1)

        def idx_issue(t, b):
            off = (base + t) * ch
            pltpu.async_copy(edges.at[pl.ds(gbase + off, ch)], gib[b], isem[b])
            pltpu.async_copy(edges.at[pl.ds(sbase + off, ch)], sib[b], isem[b])

        def gather_issue(t, b):
            off = (base + t) * ch
            pltpu.make_async_copy(edges.at[pl.ds(gbase + off, ch)], gib[b], isem[b]).wait()
            pltpu.make_async_copy(edges.at[pl.ds(sbase + off, ch)], sib[b], isem[b]).wait()
            pltpu.async_copy(table.at[gib[b]], rb[b], gsem[b])

        def body(j, carry):
            for b in range(NBUF):
                i = j * NBUF + b

                @pl.when(j > 0)
                def _free():
                    scatter_drain(b)

                idx_issue(i, b)

                @pl.when(i >= GOFF)
                def _g():
                    gather_issue(i - GOFF, (b - GOFF) % NBUF)

                @pl.when(i >= SOFF)
                def _s():
                    scatter_issue((b - SOFF) % NBUF)

            return carry

        lax.fori_loop(0, npt // NBUF, body, 0)
        for k in range(GOFF):
            gather_issue(npt - GOFF + k, (NBUF - GOFF + k) % NBUF)
        for k in range(SOFF):
            scatter_drain((NBUF - SOFF + k - 1) % NBUF)
            scatter_issue((NBUF - SOFF + k) % NBUF)
        scatter_drain(NBUF - 1)

        plsc.subcore_barrier()

        obase = s * ROWS_O
        pltpu.sync_copy(acc.at[pl.ds(obase, ROWS_O)], out.at[c, pl.ds(obase, ROWS_O)])

        @pl.when(s == NS - 1)
        def _tail():
            tb = NS * ROWS_O
            pltpu.sync_copy(acc.at[pl.ds(tb, N - tb)], out.at[c, pl.ds(tb, N - tb)])

    return seg


N_T = N + NS
_seg_neighbor = _make_seg(P1[0], P1[1], CH, N)
_seg_sub_f = _make_seg(P2[0], P2[1], CH, N_T)
_seg_sub_r = _make_seg(P2[0], P2[1], CH, N_T, swap=True)


BM = 2000
BMT = 2504


def _mm_a_body(x_ref, p_ref, wr_ref, wn_ref, b_ref, o_ref):
    agg = p_ref[0] + p_ref[1]
    o_ref[...] = (
        jnp.dot(x_ref[...], wr_ref[...], preferred_element_type=jnp.float32)
        + jnp.dot(agg, wn_ref[...], preferred_element_type=jnp.float32)
        + b_ref[...]
    )


def _mm_b_body(p_ref, w1_ref, b1_ref, w2_ref, o_ref):
    t = (
        jnp.dot(p_ref[0] + p_ref[1], w1_ref[...], preferred_element_type=jnp.float32)
        + b1_ref[...]
    )
    o_ref[...] = jnp.dot(t, w2_ref[...], preferred_element_type=jnp.float32)


def _mm_c_body(h_ref, q_ref, b2_ref, o_ref):
    o_ref[...] = h_ref[...] + q_ref[0] + q_ref[1] + b2_ref[...]


_ROW = pl.BlockSpec((BM, D), lambda i: (i, 0))
_PART = pl.BlockSpec((NC, BM, D), lambda i: (0, i, 0))
_ROWT = pl.BlockSpec((BMT, D), lambda i: (i, 0))
_PARTT = pl.BlockSpec((NC, BMT, D), lambda i: (0, i, 0))
_WMAT = pl.BlockSpec((D, D), lambda i: (0, 0))
_BVEC = pl.BlockSpec((1, D), lambda i: (0, 0))
_OUTF = jax.ShapeDtypeStruct((N, D), jnp.float32)
_OUTT = jax.ShapeDtypeStruct((N_T, D), jnp.float32)


def _mm_a(x, p, wr, wn, b):
    return pl.pallas_call(
        _mm_a_body,
        grid=(N_T // BMT,),
        in_specs=[_ROWT, _PARTT, _WMAT, _WMAT, _BVEC],
        out_specs=_ROWT,
        out_shape=_OUTT,
    )(x, p, wr, wn, b)


def _mm_b(p, w1, b1, w2):
    return pl.pallas_call(
        _mm_b_body,
        grid=(N_T // BMT,),
        in_specs=[_PARTT, _WMAT, _BVEC, _WMAT],
        out_specs=_ROWT,
        out_shape=_OUTT,
    )(p, w1, b1, w2)


def _mm_c(h, q, b2):
    return pl.pallas_call(
        _mm_c_body,
        grid=(N // BM,),
        in_specs=[_ROW, _PART, _BVEC],
        out_specs=_ROW,
        out_shape=_OUTF,
    )(h, q, b2)


def _pad_const(npts, n_edges, both_dummy):
    pad = NS * (npts[0] + npts[1]) * CH - n_edges
    dummies = (DUMMY + np.arange(pad, dtype=np.int32) % NS).astype(np.int32)
    g = dummies if both_dummy else np.zeros((pad,), np.int32)
    return np.stack([g, dummies])


_PADN = _pad_const(P1, 320000, False)
_PADS = _pad_const(P2, 100000, True)


def kernel(x, neighbor_edge_index, substructures_edge_index, W_root, W_nb, b_mn, W_n2s, b_n2s, W_s2n, b_s2n):
    sei = substructures_edge_index[0]
    e1 = jnp.concatenate([neighbor_edge_index, _PADN], axis=1).reshape(-1)
    e2 = jnp.concatenate([sei, _PADS], axis=1).reshape(-1)

    b_mn2 = b_mn.reshape(1, D)
    b_n2s2 = b_n2s.reshape(1, D)
    b_s2n2 = b_s2n.reshape(1, D)

    agg = _seg_neighbor(x, e1)
    h = _mm_a(x, agg, W_root, W_nb, b_mn2)
    sub = _seg_sub_f(h, e2)
    t2 = _mm_b(sub, W_n2s, b_n2s2, W_s2n)
    q = _seg_sub_r(t2, e2)
    return _mm_c(h, q, b_s2n2)
